# baseline (speedup 1.0000x reference)
import functools

import jax
import jax.numpy as jnp
from jax import lax
from jax.experimental import pallas as pl
from jax.experimental.pallas import tpu as pltpu
from jax.experimental.pallas import tpu_sc as plsc

D = 128
NPAD = 10112
CHUNK = 128
NTILES = 32
ROWS_PER_TILE = NPAD // 16
BN = 632
N0CH = 155
N1CH = 3
TCH = 16 * (N0CH + N1CH)


def _make_sc_agg():
    mesh = plsc.VectorSubcoreMesh(core_axis_name="c", subcore_axis_name="s")

    scratch = [
        pltpu.VMEM((3, 2, CHUNK), jnp.int32),
        pltpu.VMEM((3, CHUNK, D), jnp.float32),
        pltpu.VMEM_SHARED((NPAD, D), jnp.float32),
        pltpu.SemaphoreType.DMA,
        pltpu.SemaphoreType.DMA,
    ]

    def body(x_hbm, edges_hbm, zfull_hbm, agg_out, idx_v, rows_v, agg_sh,
             isem, gsem):
        c = lax.axis_index("c")
        s = lax.axis_index("s")
        r0 = s * ROWS_PER_TILE
        start = jnp.where(c == 0, s * N0CH, 16 * N0CH + s * N1CH)
        cnt = jnp.where(c == 0, N0CH, N1CH)

        pltpu.sync_copy(zfull_hbm.at[pl.ds(r0, ROWS_PER_TILE)],
                        agg_sh.at[pl.ds(r0, ROWS_PER_TILE)])
        plsc.subcore_barrier()

        pltpu.sync_copy(edges_hbm.at[start], idx_v.at[0])

        @pl.when(cnt > 1)
        def _():
            pltpu.async_copy(edges_hbm.at[start + 1], idx_v.at[1], isem)

        @pl.when(cnt > 2)
        def _():
            pltpu.async_copy(edges_hbm.at[start + 2], idx_v.at[2], isem)

        pltpu.async_copy(x_hbm.at[idx_v.at[0, 0]], rows_v.at[0], gsem)

        def chunk_body(j, _):
            b = lax.rem(j, 3)

            @pl.when(j + 1 < cnt)
            def _():
                nb = lax.rem(j + 1, 3)
                pltpu.make_async_copy(edges_hbm.at[start], idx_v.at[0],
                                      isem).wait()
                pltpu.async_copy(x_hbm.at[idx_v.at[nb, 0]], rows_v.at[nb],
                                 gsem)

            pltpu.make_async_copy(x_hbm.at[idx_v.at[b, 0]], rows_v.at[b],
                                  gsem).wait()
            pltpu.sync_copy(rows_v.at[b], agg_sh.at[idx_v.at[b, 1]], add=True)

            @pl.when(j + 3 < cnt)
            def _():
                pltpu.async_copy(edges_hbm.at[start + j + 3], idx_v.at[b],
                                 isem)
            return 0
        lax.fori_loop(0, cnt, chunk_body, 0)
        plsc.subcore_barrier()

        pltpu.sync_copy(agg_sh.at[pl.ds(r0, ROWS_PER_TILE)],
                        agg_out.at[c, pl.ds(r0, ROWS_PER_TILE)])

    return pl.kernel(body,
                     out_type=jax.ShapeDtypeStruct((2, NPAD, D), jnp.float32),
                     mesh=mesh, scratch_types=scratch)


def _make_sc_deg(nch):
    mesh = plsc.VectorSubcoreMesh(core_axis_name="c", subcore_axis_name="s")

    scratch = [
        pltpu.VMEM((nch, CHUNK), jnp.int32),
        pltpu.VMEM((CHUNK, D), jnp.float32),
        pltpu.VMEM_SHARED((NPAD, D), jnp.float32),
    ]

    def body(dst_hbm, ones_hbm, zfull_hbm, deg_out, dst_v, ones_v, deg_sh):
        c = lax.axis_index("c")
        s = lax.axis_index("s")
        wid = c * 16 + s
        r0 = s * ROWS_PER_TILE

        pltpu.sync_copy(dst_hbm.at[wid], dst_v)
        pltpu.sync_copy(ones_hbm, ones_v)
        pltpu.sync_copy(zfull_hbm.at[pl.ds(r0, ROWS_PER_TILE)],
                        deg_sh.at[pl.ds(r0, ROWS_PER_TILE)])
        plsc.subcore_barrier()

        def chunk_body(j, _):
            pltpu.sync_copy(ones_v, deg_sh.at[dst_v.at[j]], add=True)
            return 0
        lax.fori_loop(0, nch, chunk_body, 0)
        plsc.subcore_barrier()

        pltpu.sync_copy(deg_sh.at[pl.ds(r0, ROWS_PER_TILE)],
                        deg_out.at[c, pl.ds(r0, ROWS_PER_TILE)])

    return pl.kernel(body,
                     out_type=jax.ShapeDtypeStruct((2, NPAD, D), jnp.float32),
                     mesh=mesh, scratch_types=scratch)


def _tc_layer_body(p_ref, dg_ref, x_ref, wl_ref, bl_ref, wr_ref, o_ref):
    deg = dg_ref[0, :, 0:1] + dg_ref[1, :, 0:1]
    inv = 1.0 / jnp.maximum(deg, 1.0)
    agg = (p_ref[0] + p_ref[1]) * inv
    dn = (((1,), (1,)), ((), ()))
    y = lax.dot_general(agg, wl_ref[...], dn, preferred_element_type=jnp.float32)
    y += lax.dot_general(x_ref[...], wr_ref[...], dn, preferred_element_type=jnp.float32)
    y += bl_ref[...]
    o_ref[...] = jnp.maximum(y, 0.0)


def _tc_layer(p, degp, x, wl, bl, wr):
    grid = NPAD // BN
    return pl.pallas_call(
        _tc_layer_body,
        grid=(grid,),
        in_specs=[
            pl.BlockSpec((2, BN, D), lambda i: (0, i, 0)),
            pl.BlockSpec((2, BN, D), lambda i: (0, i, 0)),
            pl.BlockSpec((BN, D), lambda i: (i, 0)),
            pl.BlockSpec((D, D), lambda i: (0, 0)),
            pl.BlockSpec((1, D), lambda i: (0, 0)),
            pl.BlockSpec((D, D), lambda i: (0, 0)),
        ],
        out_specs=pl.BlockSpec((BN, D), lambda i: (i, 0)),
        out_shape=jax.ShapeDtypeStruct((NPAD, D), jnp.float32),
    )(p, degp, x, wl, bl.reshape(1, D), wr)


def _tc_readout_body(x1_ref, x2_ref, x3_ref, w1_ref, w2_ref, w3_ref, b_ref,
                     o_ref):
    dn = (((1,), (1,)), ((), ()))
    z = lax.dot_general(x1_ref[...], w1_ref[...], dn, preferred_element_type=jnp.float32)
    z += lax.dot_general(x2_ref[...], w2_ref[...], dn, preferred_element_type=jnp.float32)
    z += lax.dot_general(x3_ref[...], w3_ref[...], dn, preferred_element_type=jnp.float32)
    z += b_ref[...]
    z -= jnp.max(z, axis=-1, keepdims=True)
    o_ref[...] = z - jnp.log(jnp.sum(jnp.exp(z), axis=-1, keepdims=True))


def _tc_readout(x1, x2, x3, w1, w2, w3, blin):
    grid = NPAD // BN
    wspec = pl.BlockSpec((D, D), lambda i: (0, 0))
    xspec = pl.BlockSpec((BN, D), lambda i: (i, 0))
    return pl.pallas_call(
        _tc_readout_body,
        grid=(grid,),
        in_specs=[xspec, xspec, xspec, wspec, wspec, wspec,
                  pl.BlockSpec((1, D), lambda i: (0, 0))],
        out_specs=xspec,
        out_shape=jax.ShapeDtypeStruct((NPAD, D), jnp.float32),
    )(x1, x2, x3, w1, w2, w3, blin.reshape(1, D))


def kernel(x0, edge_index, Wl1, bl1, Wr1, Wl2, bl2, Wr2, Wl3, bl3, Wr3,
           Wlin, blin):
    n, _ = x0.shape
    e = edge_index.shape[1]
    epad = TCH * CHUNK
    nch_deg = TCH // NTILES

    src = edge_index[0]
    dst = edge_index[1]
    srcp = jnp.concatenate([src, jnp.zeros((epad - e,), jnp.int32)])
    dstp = jnp.concatenate([dst, jnp.full((epad - e,), n, jnp.int32)])
    dst_t = dstp.reshape(NTILES, nch_deg, CHUNK)
    edges_t = jnp.concatenate(
        [srcp.reshape(TCH, 1, CHUNK), dstp.reshape(TCH, 1, CHUNK)], axis=1)

    xp = jnp.pad(x0, ((0, NPAD - n), (0, 0)))
    zfull = jnp.zeros((NPAD, D), jnp.float32)
    ones = jnp.ones((CHUNK, D), jnp.float32)

    sc_agg = _make_sc_agg()
    sc_deg = _make_sc_deg(nch_deg)

    degp = sc_deg(dst_t, ones, zfull)
    p1 = sc_agg(xp, edges_t, zfull)
    x1 = _tc_layer(p1, degp, xp, Wl1, bl1, Wr1)
    p2 = sc_agg(x1, edges_t, zfull)
    x2 = _tc_layer(p2, degp, x1, Wl2, bl2, Wr2)
    p3 = sc_agg(x2, edges_t, zfull)
    x3 = _tc_layer(p3, degp, x2, Wl3, bl3, Wr3)

    w1 = Wlin[:, :D]
    w2 = Wlin[:, D:2 * D]
    w3 = Wlin[:, 2 * D:]
    out = _tc_readout(x1, x2, x3, w1, w2, w3, blin)
    return out[:n]


# speedup vs baseline: 1.1583x; 1.1583x over previous
import functools

import jax
import jax.numpy as jnp
from jax import lax
from jax.experimental import pallas as pl
from jax.experimental.pallas import tpu as pltpu
from jax.experimental.pallas import tpu_sc as plsc

D = 128
NPAD = 10112
CHUNK = 128
NTILES = 32
ROWS_PER_TILE = NPAD // 16
BN = 632
N0CH = 150
N1CH = 8
TCH = 16 * (N0CH + N1CH)


def _make_sc_agg():
    mesh = plsc.VectorSubcoreMesh(core_axis_name="c", subcore_axis_name="s")

    scratch = [
        pltpu.VMEM((3, 2, CHUNK), jnp.int32),
        pltpu.VMEM((3, CHUNK, D), jnp.float32),
        pltpu.VMEM_SHARED((NPAD, D), jnp.float32),
        pltpu.SemaphoreType.DMA,
        pltpu.SemaphoreType.DMA,
    ]

    def body(x_hbm, edges_hbm, zfull_hbm, agg_out, idx_v, rows_v, agg_sh,
             isem, gsem):
        c = lax.axis_index("c")
        s = lax.axis_index("s")
        r0 = s * ROWS_PER_TILE
        start = jnp.where(c == 0, s * N0CH, 16 * N0CH + s * N1CH)
        cnt = jnp.where(c == 0, N0CH, N1CH)

        pltpu.sync_copy(zfull_hbm.at[pl.ds(r0, ROWS_PER_TILE)],
                        agg_sh.at[pl.ds(r0, ROWS_PER_TILE)])
        plsc.subcore_barrier()

        pltpu.sync_copy(edges_hbm.at[start], idx_v.at[0])

        @pl.when(cnt > 1)
        def _():
            pltpu.async_copy(edges_hbm.at[start + 1], idx_v.at[1], isem)

        @pl.when(cnt > 2)
        def _():
            pltpu.async_copy(edges_hbm.at[start + 2], idx_v.at[2], isem)

        pltpu.async_copy(x_hbm.at[idx_v.at[0, 0]], rows_v.at[0], gsem)

        def chunk_body(j, _):
            b = lax.rem(j, 3)

            @pl.when(j + 1 < cnt)
            def _():
                nb = lax.rem(j + 1, 3)
                pltpu.make_async_copy(edges_hbm.at[start], idx_v.at[0],
                                      isem).wait()
                pltpu.async_copy(x_hbm.at[idx_v.at[nb, 0]], rows_v.at[nb],
                                 gsem)

            pltpu.make_async_copy(x_hbm.at[idx_v.at[b, 0]], rows_v.at[b],
                                  gsem).wait()
            pltpu.sync_copy(rows_v.at[b], agg_sh.at[idx_v.at[b, 1]], add=True)

            @pl.when(j + 3 < cnt)
            def _():
                pltpu.async_copy(edges_hbm.at[start + j + 3], idx_v.at[b],
                                 isem)
            return 0
        lax.fori_loop(0, cnt, chunk_body, 0)
        plsc.subcore_barrier()

        pltpu.sync_copy(agg_sh.at[pl.ds(r0, ROWS_PER_TILE)],
                        agg_out.at[c, pl.ds(r0, ROWS_PER_TILE)])

    return pl.kernel(body,
                     out_type=jax.ShapeDtypeStruct((2, NPAD, D), jnp.float32),
                     mesh=mesh, scratch_types=scratch)


def _make_sc_deg(nch):
    mesh = plsc.VectorSubcoreMesh(core_axis_name="c", subcore_axis_name="s")

    scratch = [
        pltpu.VMEM((nch, CHUNK), jnp.int32),
        pltpu.VMEM((CHUNK, D), jnp.float32),
        pltpu.VMEM_SHARED((NPAD, D), jnp.float32),
    ]

    def body(dst_hbm, ones_hbm, zfull_hbm, deg_out, dst_v, ones_v, deg_sh):
        c = lax.axis_index("c")
        s = lax.axis_index("s")
        wid = c * 16 + s
        r0 = s * ROWS_PER_TILE

        pltpu.sync_copy(dst_hbm.at[wid], dst_v)
        pltpu.sync_copy(ones_hbm, ones_v)
        pltpu.sync_copy(zfull_hbm.at[pl.ds(r0, ROWS_PER_TILE)],
                        deg_sh.at[pl.ds(r0, ROWS_PER_TILE)])
        plsc.subcore_barrier()

        def chunk_body(j, _):
            pltpu.sync_copy(ones_v, deg_sh.at[dst_v.at[j]], add=True)
            return 0
        lax.fori_loop(0, nch, chunk_body, 0)
        plsc.subcore_barrier()

        pltpu.sync_copy(deg_sh.at[pl.ds(r0, ROWS_PER_TILE)],
                        deg_out.at[c, pl.ds(r0, ROWS_PER_TILE)])

    return pl.kernel(body,
                     out_type=jax.ShapeDtypeStruct((2, NPAD, D), jnp.float32),
                     mesh=mesh, scratch_types=scratch)


_DN = (((1,), (1,)), ((), ()))
_WSPEC = pl.BlockSpec((D, D), lambda i: (0, 0))
_BSPEC = pl.BlockSpec((1, D), lambda i: (0, 0))
_XSPEC = pl.BlockSpec((BN, D), lambda i: (i, 0))
_PSPEC = pl.BlockSpec((2, BN, D), lambda i: (0, i, 0))


def _tc_pre_body(x_ref, wr_ref, bl_ref, o_ref):
    o_ref[...] = (lax.dot_general(x_ref[...], wr_ref[...], _DN,
                                  preferred_element_type=jnp.float32)
                  + bl_ref[...])


def _tc_pre(x, wr, bl):
    return pl.pallas_call(
        _tc_pre_body,
        grid=(NPAD // BN,),
        in_specs=[_XSPEC, _WSPEC, _BSPEC],
        out_specs=_XSPEC,
        out_shape=jax.ShapeDtypeStruct((NPAD, D), jnp.float32),
    )(x, wr, bl.reshape(1, D))


def _agg_from(p_ref, dg_ref):
    deg = dg_ref[0, :, 0:1] + dg_ref[1, :, 0:1]
    inv = 1.0 / jnp.maximum(deg, 1.0)
    return (p_ref[0] + p_ref[1]) * inv


def _tc_post_body(p_ref, dg_ref, r_ref, wl_ref, o_ref):
    y = lax.dot_general(_agg_from(p_ref, dg_ref), wl_ref[...], _DN,
                        preferred_element_type=jnp.float32) + r_ref[...]
    o_ref[...] = jnp.maximum(y, 0.0)


def _tc_post(p, degp, r, wl):
    return pl.pallas_call(
        _tc_post_body,
        grid=(NPAD // BN,),
        in_specs=[_PSPEC, _PSPEC, _XSPEC, _WSPEC],
        out_specs=_XSPEC,
        out_shape=jax.ShapeDtypeStruct((NPAD, D), jnp.float32),
    )(p, degp, r, wl)


def _tc_ro_pre_body(x1_ref, x2_ref, w1_ref, w2_ref, b_ref, o_ref):
    z = lax.dot_general(x1_ref[...], w1_ref[...], _DN,
                        preferred_element_type=jnp.float32)
    z += lax.dot_general(x2_ref[...], w2_ref[...], _DN,
                         preferred_element_type=jnp.float32)
    o_ref[...] = z + b_ref[...]


def _tc_ro_pre(x1, x2, w1, w2, blin):
    return pl.pallas_call(
        _tc_ro_pre_body,
        grid=(NPAD // BN,),
        in_specs=[_XSPEC, _XSPEC, _WSPEC, _WSPEC, _BSPEC],
        out_specs=_XSPEC,
        out_shape=jax.ShapeDtypeStruct((NPAD, D), jnp.float32),
    )(x1, x2, w1, w2, blin.reshape(1, D))


def _tc_ro_post_body(p_ref, dg_ref, r_ref, wl_ref, z_ref, w3_ref, o_ref):
    x3 = lax.dot_general(_agg_from(p_ref, dg_ref), wl_ref[...], _DN,
                         preferred_element_type=jnp.float32) + r_ref[...]
    x3 = jnp.maximum(x3, 0.0)
    z = z_ref[...] + lax.dot_general(x3, w3_ref[...], _DN,
                                     preferred_element_type=jnp.float32)
    z -= jnp.max(z, axis=-1, keepdims=True)
    o_ref[...] = z - jnp.log(jnp.sum(jnp.exp(z), axis=-1, keepdims=True))


def _tc_ro_post(p3, degp, r3, wl3, z12, w3):
    return pl.pallas_call(
        _tc_ro_post_body,
        grid=(NPAD // BN,),
        in_specs=[_PSPEC, _PSPEC, _XSPEC, _WSPEC, _XSPEC, _WSPEC],
        out_specs=_XSPEC,
        out_shape=jax.ShapeDtypeStruct((NPAD, D), jnp.float32),
    )(p3, degp, r3, wl3, z12, w3)


def kernel(x0, edge_index, Wl1, bl1, Wr1, Wl2, bl2, Wr2, Wl3, bl3, Wr3,
           Wlin, blin):
    n, _ = x0.shape
    e = edge_index.shape[1]
    epad = TCH * CHUNK
    nch_deg = TCH // NTILES

    src = edge_index[0]
    dst = edge_index[1]
    srcp = jnp.concatenate([src, jnp.zeros((epad - e,), jnp.int32)])
    dstp = jnp.concatenate([dst, jnp.full((epad - e,), n, jnp.int32)])
    dst_t = dstp.reshape(NTILES, nch_deg, CHUNK)
    edges_t = jnp.concatenate(
        [srcp.reshape(TCH, 1, CHUNK), dstp.reshape(TCH, 1, CHUNK)], axis=1)

    xp = jnp.pad(x0, ((0, NPAD - n), (0, 0)))
    zfull = jnp.zeros((NPAD, D), jnp.float32)
    ones = jnp.ones((CHUNK, D), jnp.float32)

    sc_agg = _make_sc_agg()
    sc_deg = _make_sc_deg(nch_deg)

    w1 = Wlin[:, :D]
    w2 = Wlin[:, D:2 * D]
    w3 = Wlin[:, 2 * D:]

    degp = sc_deg(dst_t, ones, zfull)
    p1 = sc_agg(xp, edges_t, zfull)
    r1 = _tc_pre(xp, Wr1, bl1)
    x1 = _tc_post(p1, degp, r1, Wl1)
    p2 = sc_agg(x1, edges_t, zfull)
    r2 = _tc_pre(x1, Wr2, bl2)
    x2 = _tc_post(p2, degp, r2, Wl2)
    p3 = sc_agg(x2, edges_t, zfull)
    r3 = _tc_pre(x2, Wr3, bl3)
    z12 = _tc_ro_pre(x1, x2, w1, w2, blin)
    out = _tc_ro_post(p3, degp, r3, Wl3, z12, w3)
    return out[:n]
